# baseline (speedup 1.0000x reference)
import functools

import jax
import jax.numpy as jnp
from jax import lax
from jax.experimental import pallas as pl
from jax.experimental.pallas import tpu as pltpu
from jax.experimental.pallas import tpu_sc as plsc

_HIDDEN = 64
_GRP = 128
_NC, _NS = 2, 16
_NW = _NC * _NS
_LANES = 16


@functools.cache
def _make_gather(bs, seq):
    nbt = bs // _GRP
    assert nbt == _NW
    nst = seq // 8
    assert nst * 8 == seq and seq % 2 == 0
    mesh = plsc.VectorSubcoreMesh(core_axis_name="c", subcore_axis_name="s")

    @functools.partial(
        pl.kernel,
        out_type=jax.ShapeDtypeStruct(
            (seq, _HIDDEN // 8, nbt, 8, _GRP), jnp.float32
        ),
        mesh=mesh,
        scratch_types=[
            pltpu.VMEM((2, _GRP), jnp.int32),
            pltpu.VMEM((2, _GRP), jnp.int32),
            pltpu.VMEM((2, _GRP, _HIDDEN), jnp.float32),
            pltpu.VMEM((2, _GRP, _HIDDEN), jnp.float32),
            pltpu.VMEM((2, _HIDDEN // 8, 8, _GRP), jnp.float32),
            pltpu.VMEM((2, _HIDDEN // 8, 8, _GRP), jnp.float32),
            pltpu.SemaphoreType.DMA,
            pltpu.SemaphoreType.DMA,
        ],
        compiler_params=pltpu.CompilerParams(
            use_tc_tiling_on_sc=False, needs_layout_passes=False
        ),
    )
    def gather(table_hbm, ids_hbm, out_hbm,
               idx0, idx1, rows0, rows1, t0, t1, sem0, sem1):
        w = lax.axis_index("s") * _NC + lax.axis_index("c")
        idx_v = (idx0, idx1)
        rows_v = (rows0, rows1)
        tb = (t0, t1)
        sems = (sem0, sem1)

        def fire(c, b):
            for j in range(2):
                s = 2 * c + j
                pltpu.sync_copy(ids_hbm.at[s // 8, w, s % 8],
                                idx_v[b].at[j])
                pltpu.async_copy(
                    table_hbm.at[idx_v[b].at[j]], rows_v[b].at[j], sems[b]
                )

        def drain(b):
            for j in range(2):
                pltpu.make_async_copy(
                    table_hbm.at[pl.ds(0, _GRP)], rows_v[b].at[j], sems[b]
                ).wait()

        def transpose_store(c, b):
            for j in range(2):
                s = 2 * c + j
                src = rows_v[b].at[j]
                dst = tb[b].at[j]

                @pl.loop(0, _HIDDEN)
                def _(h):
                    hcol = jnp.full((_LANES,), h, jnp.int32)
                    for c0 in range(_GRP // _LANES):
                        rows = c0 * _LANES + lax.iota(jnp.int32, _LANES)
                        vec = plsc.load_gather(src, [rows, hcol])
                        dst[h // 8, h % 8, pl.ds(c0 * _LANES, _LANES)] = vec

                pltpu.sync_copy(tb[b].at[j], out_hbm.at[s, :, w])

        fire(0, 0)

        @pl.loop(0, seq // 2, step=2)
        def _(c):
            fire(c + 1, 1)
            drain(0)
            transpose_store(c, 0)

            @pl.when(c + 2 < seq // 2)
            def _():
                fire(c + 2, 0)

            drain(1)
            transpose_store(c + 1, 1)

    return gather


def _ws_body(ws_ref, out_ref):
    out_ref[...] = jnp.broadcast_to(ws_ref[...], out_ref.shape)


@functools.cache
def _make_ws_broadcast(bs, w):
    blk = 256
    assert bs % blk == 0
    return pl.pallas_call(
        _ws_body,
        grid=(bs // blk,),
        in_specs=[pl.BlockSpec((1, w, _HIDDEN), lambda i: (0, 0, 0))],
        out_specs=pl.BlockSpec((blk, w, _HIDDEN), lambda i: (i, 0, 0)),
        out_shape=jax.ShapeDtypeStruct((bs, w, _HIDDEN), jnp.float32),
    )


def kernel(input_ids, attention_mask, init_workspace, emb_table):
    bs, seq = input_ids.shape
    ids4 = (
        input_ids.T.reshape(seq // 8, 8, bs // _GRP, _GRP)
        .transpose(0, 2, 1, 3)
    )
    out5 = _make_gather(bs, seq)(emb_table, ids4)
    embeddings = jnp.transpose(out5, (2, 4, 0, 1, 3)).reshape(bs, seq, _HIDDEN)
    workspace = _make_ws_broadcast(bs, init_workspace.shape[1])(init_workspace)
    return (workspace, embeddings)
